# baseline (device time: 82474 ns/iter reference)
import jax
import jax.numpy as jnp
from jax import lax
from jax.experimental import pallas as pl
from jax.experimental.pallas import tpu as pltpu

N_DEV = 8
B, SQ, SKV = 2, 512, 512
HQ_LOC, DH = 8, 64
EMB = 768
ROWS = B * SQ
CHUNK = ROWS // N_DEV
N_RS = N_DEV - 1
N_AG = N_DEV - 1
N_STEPS = N_RS + N_AG


def _allreduce_body(p_ref, out_ref, comm, sbuf, send_sems, recv_sems):
    my = lax.axis_index("i")
    left = (my + N_DEV - 1) % N_DEV
    right = (my + 1) % N_DEV

    barrier = pltpu.get_barrier_semaphore()
    for nbr in (left, right):
        pl.semaphore_signal(
            barrier, inc=1, device_id=(nbr,), device_id_type=pl.DeviceIdType.MESH
        )
    pl.semaphore_wait(barrier, 2)

    def rows(c):
        return pl.ds(c * CHUNK, CHUNK)

    sbuf[0] = p_ref[rows(my), :]
    for s in range(N_RS):
        rdma = pltpu.make_async_remote_copy(
            src_ref=sbuf.at[s],
            dst_ref=comm.at[s],
            send_sem=send_sems.at[s],
            recv_sem=recv_sems.at[s],
            device_id=(right,),
            device_id_type=pl.DeviceIdType.MESH,
        )
        rdma.start()
        rdma.wait()
        cr = (my + N_DEV - s - 1) % N_DEV
        acc = comm[s] + p_ref[rows(cr), :]
        if s < N_RS - 1:
            sbuf[s + 1] = acc
        else:
            out_ref[rows((my + 1) % N_DEV), :] = acc
            sbuf[N_RS] = acc

    for s in range(N_AG):
        src = sbuf.at[N_RS] if s == 0 else comm.at[N_RS + s - 1]
        rdma = pltpu.make_async_remote_copy(
            src_ref=src,
            dst_ref=comm.at[N_RS + s],
            send_sem=send_sems.at[N_RS + s],
            recv_sem=recv_sems.at[N_RS + s],
            device_id=(right,),
            device_id_type=pl.DeviceIdType.MESH,
        )
        rdma.start()
        rdma.wait()
        g = (my + N_DEV - s) % N_DEV
        out_ref[rows(g), :] = comm[N_RS + s]


def _ring_allreduce(partial2d):
    return pl.pallas_call(
        _allreduce_body,
        out_shape=jax.ShapeDtypeStruct((ROWS, EMB), jnp.bfloat16),
        in_specs=[pl.BlockSpec(memory_space=pltpu.VMEM)],
        out_specs=pl.BlockSpec(memory_space=pltpu.VMEM),
        scratch_shapes=[
            pltpu.VMEM((N_STEPS, CHUNK, EMB), jnp.bfloat16),
            pltpu.VMEM((N_RS + 1, CHUNK, EMB), jnp.bfloat16),
            pltpu.SemaphoreType.DMA((N_STEPS,)),
            pltpu.SemaphoreType.DMA((N_STEPS,)),
        ],
        compiler_params=pltpu.CompilerParams(collective_id=0),
    )(partial2d)


def kernel(x, Wq, K_ext, V_ext, Wo):
    my = lax.axis_index("i")
    xb = x.astype(jnp.bfloat16)

    q = jnp.einsum(
        "bse,eh->bsh", xb, Wq.astype(jnp.bfloat16),
        preferred_element_type=jnp.float32,
    ).reshape(B, SQ, HQ_LOC, DH).astype(jnp.bfloat16)

    k = lax.dynamic_slice_in_dim(K_ext, my * HQ_LOC, HQ_LOC, axis=2)
    v = lax.dynamic_slice_in_dim(V_ext, my * HQ_LOC, HQ_LOC, axis=2)

    scores = jnp.einsum(
        "bihd,bjhd->bhij", q, k.astype(jnp.bfloat16),
        preferred_element_type=jnp.float32,
    ) * 0.125
    qb = jnp.arange(SQ)[:, None] // 64
    kb = jnp.arange(SKV)[None, :] // 64
    mask = kb <= qb
    scores = jnp.where(mask[None, None], scores, -1e9)
    w = jax.nn.softmax(scores, axis=-1).astype(jnp.bfloat16)

    ctx = jnp.einsum(
        "bhij,bjhd->bihd", w, v.astype(jnp.bfloat16),
        preferred_element_type=jnp.float32,
    ).reshape(B, SQ, HQ_LOC * DH).astype(jnp.bfloat16)

    partial = jnp.einsum(
        "bsh,he->bse", ctx, Wo.astype(jnp.bfloat16),
        preferred_element_type=jnp.float32,
    )

    out = _ring_allreduce(partial.astype(jnp.bfloat16).reshape(ROWS, EMB))
    return out.reshape(B, SQ, EMB).astype(jnp.float32)


# device time: 67237 ns/iter; 1.2266x vs baseline; 1.2266x over previous
import jax
import jax.numpy as jnp
from jax import lax
from jax.experimental import pallas as pl
from jax.experimental.pallas import tpu as pltpu

N_DEV = 8
B, SQ, SKV = 2, 512, 512
HQ_LOC, DH = 8, 64
EMB = 768
ROWS = B * SQ
R = ROWS // N_DEV

RS_MASKS = (4, 2, 1)
AG_MASKS = (1, 2, 4)
RS_HI = {4: 0, 2: 4, 1: 6}
AG_LO = {1: 7, 2: 6, 4: 4}
RBUF_OFF = {4: 0, 2: 4 * R, 1: 6 * R}


def _allreduce_body(p_ref, out_ref, rbuf, send_sems, recv_sems):
    my = lax.axis_index("i")
    v = my ^ ((my >> 1) & 1)

    def phys(u):
        return u ^ ((u >> 1) & 1)

    partners = {m: phys(v ^ m) for m in (1, 2, 4)}

    barrier = pltpu.get_barrier_semaphore()
    for m in (1, 2, 4):
        pl.semaphore_signal(
            barrier, inc=1,
            device_id=(partners[m],), device_id_type=pl.DeviceIdType.MESH,
        )
    pl.semaphore_wait(barrier, 3)

    out_ref[...] = p_ref[...]

    for step, m in enumerate(RS_MASKS):
        sz = m * R
        base = (v & RS_HI[m]) * R
        keep = base + (v & m) * R
        send = base + ((v ^ m) & m) * R
        rdma = pltpu.make_async_remote_copy(
            src_ref=out_ref.at[pl.ds(send, sz), :],
            dst_ref=rbuf.at[pl.ds(RBUF_OFF[m], sz), :],
            send_sem=send_sems.at[step],
            recv_sem=recv_sems.at[step],
            device_id=(partners[m],),
            device_id_type=pl.DeviceIdType.MESH,
        )
        rdma.start()
        rdma.wait()
        out_ref[pl.ds(keep, sz), :] = (
            out_ref[pl.ds(keep, sz), :] + rbuf[pl.ds(RBUF_OFF[m], sz), :]
        )

    for step, m in enumerate(AG_MASKS):
        sz = m * R
        send = (v & AG_LO[m]) * R
        rdma = pltpu.make_async_remote_copy(
            src_ref=out_ref.at[pl.ds(send, sz), :],
            dst_ref=out_ref.at[pl.ds(send, sz), :],
            send_sem=send_sems.at[3 + step],
            recv_sem=recv_sems.at[3 + step],
            device_id=(partners[m],),
            device_id_type=pl.DeviceIdType.MESH,
        )
        rdma.start()
        rdma.wait()


def _ring_allreduce(partial2d):
    return pl.pallas_call(
        _allreduce_body,
        out_shape=jax.ShapeDtypeStruct((ROWS, EMB), jnp.bfloat16),
        in_specs=[pl.BlockSpec(memory_space=pltpu.VMEM)],
        out_specs=pl.BlockSpec(memory_space=pltpu.VMEM),
        scratch_shapes=[
            pltpu.VMEM((7 * R, EMB), jnp.bfloat16),
            pltpu.SemaphoreType.DMA((6,)),
            pltpu.SemaphoreType.DMA((6,)),
        ],
        compiler_params=pltpu.CompilerParams(collective_id=0),
    )(partial2d)


def kernel(x, Wq, K_ext, V_ext, Wo):
    my = lax.axis_index("i")
    xb = x.astype(jnp.bfloat16)

    q = jnp.einsum(
        "bse,eh->bsh", xb, Wq.astype(jnp.bfloat16),
        preferred_element_type=jnp.float32,
    ).reshape(B, SQ, HQ_LOC, DH).astype(jnp.bfloat16)

    k = lax.dynamic_slice_in_dim(K_ext, my * HQ_LOC, HQ_LOC, axis=2)
    v = lax.dynamic_slice_in_dim(V_ext, my * HQ_LOC, HQ_LOC, axis=2)

    scores = jnp.einsum(
        "bihd,bjhd->bhij", q, k.astype(jnp.bfloat16),
        preferred_element_type=jnp.float32,
    ) * 0.125
    qb = jnp.arange(SQ)[:, None] // 64
    kb = jnp.arange(SKV)[None, :] // 64
    mask = kb <= qb
    scores = jnp.where(mask[None, None], scores, -1e9)
    w = jax.nn.softmax(scores, axis=-1).astype(jnp.bfloat16)

    ctx = jnp.einsum(
        "bhij,bjhd->bihd", w, v.astype(jnp.bfloat16),
        preferred_element_type=jnp.float32,
    ).reshape(B, SQ, HQ_LOC * DH).astype(jnp.bfloat16)

    partial = jnp.einsum(
        "bsh,he->bse", ctx, Wo.astype(jnp.bfloat16),
        preferred_element_type=jnp.float32,
    )

    out = _ring_allreduce(partial.astype(jnp.bfloat16).reshape(ROWS, EMB))
    return out.reshape(B, SQ, EMB).astype(jnp.float32)


# device time: 48834 ns/iter; 1.6889x vs baseline; 1.3768x over previous
import jax
import jax.numpy as jnp
from jax import lax
from jax.experimental import pallas as pl
from jax.experimental.pallas import tpu as pltpu

N_DEV = 8
B, SQ, SKV = 2, 512, 512
HQ_LOC, DH = 8, 64
EMB = 768
ROWS = B * SQ
R = ROWS // N_DEV
NG = 3
COLS = EMB // NG

RS_SCHED = (
    ((0, 4, ((0, 4),)), (4, 2, ((0, 2),)), (6, 1, ((0, 1),))),
    ((0, 2, ((0, 2), (4, 2))), (2, 1, ((0, 1), (4, 1))), (3, 4, ((0, 1),))),
    ((0, 1, ((0, 1), (2, 1), (4, 1), (6, 1))), (1, 4, ((0, 1), (2, 1))),
     (5, 2, ((0, 1),))),
)
AG_SCHED = (
    ((7, 1, ((0, 1),)), (6, 2, ((0, 2),)), (4, 4, ((0, 4),))),
    ((7, 4, ((0, 1),)), (3, 1, ((0, 1), (4, 1))), (2, 2, ((0, 2), (4, 2)))),
    ((7, 2, ((0, 1),)), (5, 4, ((0, 1), (2, 1))),
     (1, 1, ((0, 1), (2, 1), (4, 1), (6, 1)))),
)
RBUF_BASE = (0, 4, 6)
N_RDMA = 15


def _allreduce_body(p_ref, out_ref, rbuf, ss_rs, rs_rs, ss_ag, rs_ag):
    my = lax.axis_index("i")
    v = my ^ ((my >> 1) & 1)

    def phys(u):
        return u ^ ((u >> 1) & 1)

    partners = {m: phys(v ^ m) for m in (1, 2, 4)}

    barrier = pltpu.get_barrier_semaphore()
    for m in (1, 2, 4):
        pl.semaphore_signal(
            barrier, inc=1,
            device_id=(partners[m],), device_id_type=pl.DeviceIdType.MESH,
        )
    pl.semaphore_wait(barrier, 3)

    out_ref[...] = p_ref[...]

    def gcols(g):
        return pl.ds(g * COLS, COLS)

    rs_idx = [0]
    ag_idx = [0]

    def issue_rs(g, j):
        fixedmask, m, runs = RS_SCHED[g][j]
        send_base = (v & fixedmask) | ((v ^ m) & m)
        descs = []
        slot = RBUF_BASE[j]
        for off, n in runs:
            i = rs_idx[0]
            rs_idx[0] += 1
            rdma = pltpu.make_async_remote_copy(
                src_ref=out_ref.at[pl.ds((send_base + off) * R, n * R), gcols(g)],
                dst_ref=rbuf.at[pl.ds(slot * R, n * R), gcols(g)],
                send_sem=ss_rs.at[i],
                recv_sem=rs_rs.at[i],
                device_id=(partners[m],),
                device_id_type=pl.DeviceIdType.MESH,
            )
            rdma.start()
            descs.append(rdma)
            slot += n
        return descs

    def add_rs(g, j):
        fixedmask, m, runs = RS_SCHED[g][j]
        keep_base = (v & fixedmask) | (v & m)
        slot = RBUF_BASE[j]
        for off, n in runs:
            rows = pl.ds((keep_base + off) * R, n * R)
            out_ref[rows, gcols(g)] = (
                out_ref[rows, gcols(g)]
                + rbuf[pl.ds(slot * R, n * R), gcols(g)]
            )
            slot += n

    def issue_ag(g, j):
        validmask, m, runs = AG_SCHED[g][j]
        base = v & validmask
        descs = []
        for off, n in runs:
            i = ag_idx[0]
            ag_idx[0] += 1
            rows = pl.ds((base + off) * R, n * R)
            rdma = pltpu.make_async_remote_copy(
                src_ref=out_ref.at[rows, gcols(g)],
                dst_ref=out_ref.at[rows, gcols(g)],
                send_sem=ss_ag.at[i],
                recv_sem=rs_ag.at[i],
                device_id=(partners[m],),
                device_id_type=pl.DeviceIdType.MESH,
            )
            rdma.start()
            descs.append(rdma)
        return descs

    pend = {g: issue_rs(g, 0) for g in range(NG)}
    ag_pend = {}
    for j in range(3):
        for g in range(NG):
            for d in pend[g]:
                d.wait()
            add_rs(g, j)
            if j < 2:
                pend[g] = issue_rs(g, j + 1)
            else:
                ag_pend[g] = issue_ag(g, 0)

    for j in range(3):
        for g in range(NG):
            for d in ag_pend[g]:
                d.wait()
            if j < 2:
                ag_pend[g] = issue_ag(g, j + 1)


def _cube_allreduce(partial2d):
    return pl.pallas_call(
        _allreduce_body,
        out_shape=jax.ShapeDtypeStruct((ROWS, EMB), jnp.bfloat16),
        in_specs=[pl.BlockSpec(memory_space=pltpu.VMEM)],
        out_specs=pl.BlockSpec(memory_space=pltpu.VMEM),
        scratch_shapes=[
            pltpu.VMEM((7 * R, EMB), jnp.bfloat16),
            pltpu.SemaphoreType.DMA((N_RDMA,)),
            pltpu.SemaphoreType.DMA((N_RDMA,)),
            pltpu.SemaphoreType.DMA((N_RDMA,)),
            pltpu.SemaphoreType.DMA((N_RDMA,)),
        ],
        compiler_params=pltpu.CompilerParams(collective_id=0),
    )(partial2d)


def kernel(x, Wq, K_ext, V_ext, Wo):
    my = lax.axis_index("i")
    xb = x.astype(jnp.bfloat16)

    q = jnp.einsum(
        "bse,eh->bsh", xb, Wq.astype(jnp.bfloat16),
        preferred_element_type=jnp.float32,
    ).reshape(B, SQ, HQ_LOC, DH).astype(jnp.bfloat16)

    k = lax.dynamic_slice_in_dim(K_ext, my * HQ_LOC, HQ_LOC, axis=2)
    v = lax.dynamic_slice_in_dim(V_ext, my * HQ_LOC, HQ_LOC, axis=2)

    scores = jnp.einsum(
        "bihd,bjhd->bhij", q, k.astype(jnp.bfloat16),
        preferred_element_type=jnp.float32,
    ) * 0.125
    qb = jnp.arange(SQ)[:, None] // 64
    kb = jnp.arange(SKV)[None, :] // 64
    mask = kb <= qb
    scores = jnp.where(mask[None, None], scores, -1e9)
    w = jax.nn.softmax(scores, axis=-1).astype(jnp.bfloat16)

    ctx = jnp.einsum(
        "bhij,bjhd->bihd", w, v.astype(jnp.bfloat16),
        preferred_element_type=jnp.float32,
    ).reshape(B, SQ, HQ_LOC * DH).astype(jnp.bfloat16)

    partial = jnp.einsum(
        "bsh,he->bse", ctx, Wo.astype(jnp.bfloat16),
        preferred_element_type=jnp.float32,
    )

    out = _cube_allreduce(partial.astype(jnp.bfloat16).reshape(ROWS, EMB))
    return out.reshape(B, SQ, EMB).astype(jnp.float32)
